# baseline (speedup 1.0000x reference)
import functools

import jax
import jax.numpy as jnp
from jax import lax
from jax.experimental import pallas as pl
from jax.experimental.pallas import tpu as pltpu

_TOP_K = 8
_E = 64
_H = 2048
_N = 8192
_BLK = 1024
_CHUNK = 64


def _router_body(hs_ref, w_ref, b_ref, scores_ref, idx_ref, cnt_ref, lg_ref):
    i = pl.program_id(0)
    hs = hs_ref[...]
    w = w_ref[...]
    lg_ref[...] = (
        lax.dot_general(hs, w, (((1,), (1,)), ((), ())),
                        preferred_element_type=jnp.float32)
        + b_ref[...]
    )

    lane_f = lax.broadcasted_iota(jnp.int32, (_CHUNK, _E), 1).astype(jnp.float32)
    neg_inf = jnp.float32(-jnp.inf)

    def chunk_step(c, cnt_acc):
        base = c * _CHUNK
        logits = lg_ref[pl.ds(base, _CHUNK), :]
        avail = logits
        idx_cols = []
        top1 = None
        for k in range(_TOP_K):
            m = jnp.max(avail, axis=1, keepdims=True)
            if k == 0:
                top1 = m
            sel = jnp.min(jnp.where(avail == m, lane_f, jnp.float32(_E)),
                          axis=1, keepdims=True)
            idx_cols.append(sel)
            avail = jnp.where(lane_f == sel, neg_inf, avail)
        selected = avail == neg_inf
        num = jnp.where(selected, jnp.exp(logits - top1), 0.0)
        denom = jnp.sum(num, axis=1, keepdims=True)
        scores_ref[pl.ds(base, _CHUNK), :] = num / denom
        idx_ref[pl.ds(base, _CHUNK), :] = jnp.concatenate(
            idx_cols, axis=1).astype(jnp.int32)
        return cnt_acc + jnp.sum(selected.astype(jnp.int32), axis=0,
                                 keepdims=True)

    cnt = lax.fori_loop(0, _BLK // _CHUNK, chunk_step,
                        jnp.zeros((1, _E), jnp.int32))

    @pl.when(i == 0)
    def _init():
        cnt_ref[...] = jnp.zeros_like(cnt_ref)

    cnt_ref[...] += cnt


@jax.jit
def kernel(hidden_states, weight, bias):
    hs = hidden_states.reshape(-1, _H)
    n = hs.shape[0]
    grid = (n // _BLK,)
    scores, idx, cnt = pl.pallas_call(
        _router_body,
        grid=grid,
        in_specs=[
            pl.BlockSpec((_BLK, _H), lambda i: (i, 0)),
            pl.BlockSpec((_E, _H), lambda i: (0, 0)),
            pl.BlockSpec((1, _E), lambda i: (0, 0)),
        ],
        out_specs=[
            pl.BlockSpec((_BLK, _E), lambda i: (i, 0)),
            pl.BlockSpec((_BLK, _TOP_K), lambda i: (i, 0)),
            pl.BlockSpec((1, _E), lambda i: (0, 0)),
        ],
        out_shape=[
            jax.ShapeDtypeStruct((n, _E), jnp.float32),
            jax.ShapeDtypeStruct((n, _TOP_K), jnp.int32),
            jax.ShapeDtypeStruct((1, _E), jnp.int32),
        ],
        scratch_shapes=[pltpu.VMEM((_BLK, _E), jnp.float32)],
        compiler_params=pltpu.CompilerParams(
            dimension_semantics=("arbitrary",),
        ),
    )(hs, weight, bias.reshape(1, _E))
    return scores, idx, cnt.reshape(_E)


# speedup vs baseline: 4.6085x; 4.6085x over previous
import functools

import jax
import jax.numpy as jnp
from jax import lax
from jax.experimental import pallas as pl
from jax.experimental.pallas import tpu as pltpu

_TOP_K = 8
_E = 64
_H = 2048
_N = 8192
_BLK = 1024
_CHUNK = 64


def _router_body(hs_ref, w_ref, b_ref, scores_ref, idx_ref, cnt_ref, lg_ref):
    i = pl.program_id(0)
    hs = hs_ref[...]
    w = w_ref[...]
    lg_ref[...] = (
        lax.dot_general(hs, w, (((1,), (1,)), ((), ())),
                        preferred_element_type=jnp.float32)
        + b_ref[...]
    )

    lane_f = lax.broadcasted_iota(jnp.int32, (_CHUNK, _E), 1).astype(jnp.float32)
    neg_inf = jnp.float32(-jnp.inf)

    def chunk_step(base, cnt_acc):
        logits = lg_ref[pl.ds(base, _CHUNK), :]
        avail = logits
        idx_cols = []
        top1 = None
        for k in range(_TOP_K):
            m = jnp.max(avail, axis=1, keepdims=True)
            if k == 0:
                top1 = m
            sel = jnp.min(jnp.where(avail == m, lane_f, jnp.float32(_E)),
                          axis=1, keepdims=True)
            idx_cols.append(sel)
            avail = jnp.where(lane_f == sel, neg_inf, avail)
        selected = avail == neg_inf
        num = jnp.where(selected, jnp.exp(logits - top1), 0.0)
        denom = jnp.sum(num, axis=1, keepdims=True)
        scores_ref[pl.ds(base, _CHUNK), :] = num / denom
        idx_ref[pl.ds(base, _CHUNK), :] = jnp.concatenate(
            idx_cols, axis=1).astype(jnp.int32)
        return cnt_acc + jnp.sum(selected.astype(jnp.int32), axis=0,
                                 keepdims=True)

    cnt = jnp.zeros((1, _E), jnp.int32)
    for base in range(0, _BLK, _CHUNK):
        cnt = chunk_step(base, cnt)

    @pl.when(i == 0)
    def _init():
        cnt_ref[...] = jnp.zeros_like(cnt_ref)

    cnt_ref[...] += cnt


@jax.jit
def kernel(hidden_states, weight, bias):
    hs = hidden_states.reshape(-1, _H)
    n = hs.shape[0]
    grid = (n // _BLK,)
    scores, idx, cnt = pl.pallas_call(
        _router_body,
        grid=grid,
        in_specs=[
            pl.BlockSpec((_BLK, _H), lambda i: (i, 0)),
            pl.BlockSpec((_E, _H), lambda i: (0, 0)),
            pl.BlockSpec((1, _E), lambda i: (0, 0)),
        ],
        out_specs=[
            pl.BlockSpec((_BLK, _E), lambda i: (i, 0)),
            pl.BlockSpec((_BLK, _TOP_K), lambda i: (i, 0)),
            pl.BlockSpec((1, _E), lambda i: (0, 0)),
        ],
        out_shape=[
            jax.ShapeDtypeStruct((n, _E), jnp.float32),
            jax.ShapeDtypeStruct((n, _TOP_K), jnp.int32),
            jax.ShapeDtypeStruct((1, _E), jnp.int32),
        ],
        scratch_shapes=[pltpu.VMEM((_BLK, _E), jnp.float32)],
        compiler_params=pltpu.CompilerParams(
            dimension_semantics=("arbitrary",),
        ),
    )(hs, weight, bias.reshape(1, _E))
    return scores, idx, cnt.reshape(_E)
